# baseline (speedup 1.0000x reference)
import functools

import jax
import jax.numpy as jnp
from jax import lax
from jax.experimental import pallas as pl
from jax.experimental.pallas import tpu as pltpu
from jax.experimental.pallas import tpu_sc as plsc

D = 32
BATCH = 16384
HIST = 50
TOTAL = BATCH * HIST
NC = 2
NS = 16
NW = NC * NS
PER_W = TOTAL // NW
CHUNK = 128
NCHUNK = PER_W // CHUNK
SCALE = float(D) ** 0.5


@functools.partial(
    pl.kernel,
    mesh=plsc.VectorSubcoreMesh(core_axis_name="c", subcore_axis_name="s"),
    out_type=jax.ShapeDtypeStruct((TOTAL, D), jnp.float32),
    compiler_params=pltpu.CompilerParams(use_tc_tiling_on_sc=False),
    scratch_types=[
        pltpu.VMEM((NCHUNK, CHUNK), jnp.int32),
        pltpu.VMEM((CHUNK, D), jnp.float32),
        pltpu.VMEM((CHUNK, D), jnp.float32),
        pltpu.SemaphoreType.DMA,
        pltpu.SemaphoreType.DMA,
    ],
)
def _emb_lookup(idx_hbm, table_hbm, out_hbm, idx_v, rows_v, outb_v, gsem, osem):
    wid = lax.axis_index("s") * NC + lax.axis_index("c")
    base = wid * PER_W
    pltpu.sync_copy(idx_hbm.at[wid], idx_v)

    def chunk_body(c, _):
        pltpu.async_copy(table_hbm.at[idx_v.at[c]], rows_v, gsem).wait()

        def scale_body(i, _):
            for h in range(2):
                sl = pl.ds(h * 16, 16)
                outb_v[i, sl] = rows_v[i, sl] * SCALE
            return ()

        lax.fori_loop(0, CHUNK, scale_body, ())
        pltpu.async_copy(
            outb_v, out_hbm.at[pl.ds(base + c * CHUNK, CHUNK)], osem
        ).wait()
        return ()

    lax.fori_loop(0, NCHUNK, chunk_body, ())


def kernel(inputs, embeddings):
    idx = inputs.astype(jnp.int32).reshape(NW, NCHUNK, CHUNK)
    out = _emb_lookup(idx, embeddings)
    return out.reshape(BATCH, HIST, D)


# speedup vs baseline: 1.1363x; 1.1363x over previous
import functools

import jax
import jax.numpy as jnp
from jax import lax
from jax.experimental import pallas as pl
from jax.experimental.pallas import tpu as pltpu
from jax.experimental.pallas import tpu_sc as plsc

D = 32
BATCH = 16384
HIST = 50
TOTAL = BATCH * HIST
NC = 2
NS = 16
NW = NC * NS
PER_W = TOTAL // NW
CHUNK = 128
NCHUNK = PER_W // CHUNK
NBUF = 4
NGRP = NCHUNK // NBUF
SCALE = float(D) ** 0.5


@functools.partial(
    pl.kernel,
    mesh=plsc.VectorSubcoreMesh(core_axis_name="c", subcore_axis_name="s"),
    out_type=jax.ShapeDtypeStruct((TOTAL, D), jnp.float32),
    compiler_params=pltpu.CompilerParams(use_tc_tiling_on_sc=False),
    scratch_types=[
        pltpu.VMEM((NCHUNK, CHUNK), jnp.int32),
        pltpu.VMEM((NBUF, CHUNK, D), jnp.float32),
        pltpu.VMEM((NBUF, CHUNK, D), jnp.float32),
    ]
    + [pltpu.SemaphoreType.DMA] * (2 * NBUF),
)
def _emb_lookup(idx_hbm, table_hbm, out_hbm, idx_v, rin, rout, *sems):
    gsems = sems[:NBUF]
    osems = sems[NBUF:]
    wid = lax.axis_index("s") * NC + lax.axis_index("c")
    base = wid * PER_W
    pltpu.sync_copy(idx_hbm.at[wid], idx_v)

    def gstart(c, b):
        pltpu.async_copy(table_hbm.at[idx_v.at[c]], rin.at[b], gsems[b])

    def gwait(b):
        pltpu.make_async_copy(
            out_hbm.at[pl.ds(0, CHUNK)], rin.at[b], gsems[b]
        ).wait()

    def ostart(c, b):
        pltpu.async_copy(
            rout.at[b], out_hbm.at[pl.ds(base + c * CHUNK, CHUNK)], osems[b]
        )

    def owait(b):
        pltpu.make_async_copy(
            rout.at[b], out_hbm.at[pl.ds(0, CHUNK)], osems[b]
        ).wait()

    def scale(b):
        def srow(i, _):
            for r in range(4):
                row = i * 4 + r
                for h in range(2):
                    sl = pl.ds(h * 16, 16)
                    rout[b, row, sl] = rin[b, row, sl] * SCALE
            return ()

        lax.fori_loop(0, CHUNK // 4, srow, ())

    for b in range(NBUF):
        gstart(b, b)

    for b in range(NBUF):
        gwait(b)
        scale(b)
        ostart(b, b)
        gstart(b + NBUF, b)

    def group(g, _):
        c0 = g * NBUF
        for b in range(NBUF):
            c = c0 + b
            gwait(b)
            owait(b)
            scale(b)
            ostart(c, b)
            gstart(c + NBUF, b)
        return ()

    lax.fori_loop(1, NGRP - 1, group, ())

    c0 = (NGRP - 1) * NBUF
    for b in range(NBUF):
        gwait(b)
        owait(b)
        scale(b)
        ostart(c0 + b, b)
    for b in range(NBUF):
        owait(b)


def kernel(inputs, embeddings):
    idx = inputs.astype(jnp.int32).reshape(NW, NCHUNK, CHUNK)
    out = _emb_lookup(idx, embeddings)
    return out.reshape(BATCH, HIST, D)


# speedup vs baseline: 1.5801x; 1.3906x over previous
import functools

import jax
import jax.numpy as jnp
from jax import lax
from jax.experimental import pallas as pl
from jax.experimental.pallas import tpu as pltpu
from jax.experimental.pallas import tpu_sc as plsc

D = 32
BATCH = 16384
HIST = 50
NC = 2
NS = 16
NW = NC * NS
BPW = BATCH // NW
CB = 4
NBUF = 2
SCALE = float(D) ** 0.5


@functools.partial(
    pl.kernel,
    mesh=plsc.VectorSubcoreMesh(core_axis_name="c", subcore_axis_name="s"),
    out_type=jax.ShapeDtypeStruct((HIST, D // 8, BATCH // 128, 8, 128),
                                  jnp.float32),
    compiler_params=pltpu.CompilerParams(
        use_tc_tiling_on_sc=False, needs_layout_passes=False
    ),
    scratch_types=[
        pltpu.VMEM((BPW, HIST), jnp.int32),
        pltpu.VMEM((HIST, CB, 128), jnp.int32),
        pltpu.VMEM((NBUF, CB, 128, D), jnp.float32),
        pltpu.VMEM((NBUF, D // 8, CB, 8, 128), jnp.float32),
    ]
    + [pltpu.SemaphoreType.DMA] * (2 * NBUF),
)
def _emb_lookup(idx_hbm, table_hbm, out_hbm, idx_v, idxt_v, rin, obuf, *sems):
    gsems = sems[:NBUF]
    osems = sems[NBUF:]
    wid = lax.axis_index("s") * NC + lax.axis_index("c")
    b0 = wid * BPW
    c0 = wid * CB
    iota = lax.iota(jnp.int32, 16)

    pltpu.sync_copy(idx_hbm.at[pl.ds(b0, BPW)], idx_v)

    def idxt_body(h, _):
        hcol = jnp.full((16,), h, jnp.int32)
        for c in range(CB):
            for k in range(8):
                rows = iota + (128 * c + 16 * k)
                v = plsc.load_gather(idx_v, [rows, hcol])
                idxt_v[h, c, pl.ds(16 * k, 16)] = v
        return ()

    lax.fori_loop(0, HIST, idxt_body, ())

    def gstart(h, slot):
        for c in range(CB):
            pltpu.async_copy(
                table_hbm.at[idxt_v.at[h, c]], rin.at[slot, c], gsems[slot]
            )

    def gwait(slot):
        for c in range(CB):
            pltpu.make_async_copy(
                table_hbm.at[pl.ds(0, 128)], rin.at[slot, c], gsems[slot]
            ).wait()

    def ostart(h, slot):
        pltpu.async_copy(
            obuf.at[slot], out_hbm.at[h, :, pl.ds(c0, CB)], osems[slot]
        )

    def owait(slot):
        pltpu.make_async_copy(
            obuf.at[slot], out_hbm.at[0, :, pl.ds(c0, CB)], osems[slot]
        ).wait()

    def transpose_scale(slot):
        def col_body(j, _):
            band = j // 8
            s = j - band * 8
            dcol = jnp.full((16,), j, jnp.int32)
            for k in range(8):
                rows = iota + 16 * k
                for c in range(CB):
                    v = plsc.load_gather(rin.at[slot, c], [rows, dcol])
                    obuf[slot, band, c, s, pl.ds(16 * k, 16)] = v * SCALE
            return ()

        lax.fori_loop(0, D, col_body, ())

    for slot in range(NBUF):
        gstart(slot, slot)
    for slot in range(NBUF):
        gwait(slot)
        transpose_scale(slot)
        ostart(slot, slot)
        gstart(slot + NBUF, slot)

    def pair(g, _):
        for slot in range(NBUF):
            h = NBUF * g + slot
            gwait(slot)
            owait(slot)
            transpose_scale(slot)
            ostart(h, slot)
            gstart(h + NBUF, slot)
        return ()

    lax.fori_loop(1, HIST // NBUF - 1, pair, ())

    for slot in range(NBUF):
        h = HIST - NBUF + slot
        gwait(slot)
        owait(slot)
        transpose_scale(slot)
        ostart(h, slot)
    for slot in range(NBUF):
        owait(slot)


def kernel(inputs, embeddings):
    idx = inputs.astype(jnp.int32)
    out5 = _emb_lookup(idx, embeddings)
    return out5.transpose(2, 4, 0, 1, 3).reshape(BATCH, HIST, D)


# speedup vs baseline: 2.4194x; 1.5311x over previous
import functools

import jax
import jax.numpy as jnp
from jax import lax
from jax.experimental import pallas as pl
from jax.experimental.pallas import tpu as pltpu
from jax.experimental.pallas import tpu_sc as plsc

D = 32
BATCH = 16384
HIST = 50
NC = 2
NS = 16
NW = NC * NS
BPW = BATCH // NW
CB = 4
NBUF = 2
LPAD = 129
SCALE = float(D) ** 0.5


@functools.partial(
    pl.kernel,
    mesh=plsc.VectorSubcoreMesh(core_axis_name="c", subcore_axis_name="s"),
    out_type=jax.ShapeDtypeStruct((HIST, D // 8, BATCH // 128, 8, 128),
                                  jnp.float32),
    compiler_params=pltpu.CompilerParams(
        use_tc_tiling_on_sc=False, needs_layout_passes=False
    ),
    scratch_types=[
        pltpu.VMEM((HIST, CB, 128), jnp.int32),
        pltpu.VMEM((NBUF, CB, 128, D), jnp.float32),
        pltpu.VMEM((NBUF, D // 8, CB, 8, LPAD), jnp.float32),
    ]
    + [pltpu.SemaphoreType.DMA] * (2 * NBUF),
)
def _emb_lookup(idx_hbm, table_hbm, out_hbm, idx_v, rin, obuf, *sems):
    gsems = sems[:NBUF]
    osems = sems[NBUF:]
    wid = lax.axis_index("s") * NC + lax.axis_index("c")
    c0 = wid * CB
    iota = lax.iota(jnp.int32, 16)

    pltpu.sync_copy(idx_hbm.at[:, pl.ds(c0, CB), :], idx_v)

    def gstart(h, slot):
        for c in range(CB):
            pltpu.async_copy(
                table_hbm.at[idx_v.at[h, c]], rin.at[slot, c], gsems[slot]
            )

    def gwait(slot):
        for c in range(CB):
            pltpu.make_async_copy(
                table_hbm.at[pl.ds(0, 128)], rin.at[slot, c], gsems[slot]
            ).wait()

    def ostart(h, slot):
        pltpu.async_copy(
            obuf.at[slot, :, :, :, pl.ds(0, 128)],
            out_hbm.at[h, :, pl.ds(c0, CB)],
            osems[slot],
        )

    def owait(slot):
        pltpu.make_async_copy(
            obuf.at[slot, :, :, :, pl.ds(0, 128)],
            out_hbm.at[0, :, pl.ds(c0, CB)],
            osems[slot],
        ).wait()

    bands = [jax.lax.shift_right_logical(iota, 3) + 2 * t for t in range(2)]
    subl = jax.lax.bitwise_and(iota, jnp.full((16,), 7, jnp.int32))
    cvs = [jnp.full((16,), c, jnp.int32) for c in range(CB)]

    def transpose_scale(slot):
        def row_body(l, _):
            lv = jnp.full((16,), l, jnp.int32)
            for c in range(CB):
                for t in range(2):
                    v = rin[slot, c, l, pl.ds(16 * t, 16)] * SCALE
                    plsc.store_scatter(
                        obuf.at[slot], [bands[t], cvs[c], subl, lv], v
                    )
            return ()

        lax.fori_loop(0, 128, row_body, ())

    for slot in range(NBUF):
        gstart(slot, slot)
    for slot in range(NBUF):
        gwait(slot)
        transpose_scale(slot)
        ostart(slot, slot)
        gstart(slot + NBUF, slot)

    def pair(g, _):
        for slot in range(NBUF):
            h = NBUF * g + slot
            gwait(slot)
            owait(slot)
            transpose_scale(slot)
            ostart(h, slot)
            gstart(h + NBUF, slot)
        return ()

    lax.fori_loop(1, HIST // NBUF - 1, pair, ())

    for slot in range(NBUF):
        h = HIST - NBUF + slot
        gwait(slot)
        owait(slot)
        transpose_scale(slot)
        ostart(h, slot)
    for slot in range(NBUF):
        owait(slot)


def kernel(inputs, embeddings):
    idx = inputs.astype(jnp.int32).T.reshape(HIST, BATCH // 128, 128)
    out5 = _emb_lookup(idx, embeddings)
    return out5.transpose(2, 4, 0, 1, 3).reshape(BATCH, HIST, D)
